# baseline (speedup 1.0000x reference)
import functools

import jax
import jax.numpy as jnp
from jax import lax
from jax.experimental import pallas as pl
from jax.experimental.pallas import tpu as pltpu
from jax.experimental.pallas import tpu_sc as plsc

_EMBED_DIM = 64
_BATCH = 16384
_NC = 2
_NS = 16
_NW = _NC * _NS
_BPW = _BATCH // _NW
_CHK = 128
_NCH = _BPW // _CHK


@functools.cache
def _build():
    mesh = plsc.VectorSubcoreMesh(core_axis_name="c", subcore_axis_name="s")

    @functools.partial(
        pl.kernel,
        mesh=mesh,
        out_type=jax.ShapeDtypeStruct((_BATCH,), jnp.float32),
        compiler_params=pltpu.CompilerParams(
            needs_layout_passes=False, use_tc_tiling_on_sc=False
        ),
        scratch_types=[
            pltpu.VMEM((_NCH, _CHK), jnp.int32),
            pltpu.VMEM((_NCH, _CHK), jnp.int32),
            pltpu.VMEM((_NCH, _CHK), jnp.int32),
            pltpu.VMEM((_BPW, _EMBED_DIM), jnp.float32),
            pltpu.VMEM((_BPW, _EMBED_DIM), jnp.float32),
            pltpu.VMEM((_BPW, _EMBED_DIM), jnp.float32),
            pltpu.VMEM((_BPW,), jnp.float32),
            pltpu.SemaphoreType.DMA,
        ],
    )
    def trans_e(head_hbm, rel_hbm, tail_hbm, ent_hbm, relw_hbm, out_hbm,
                hidx, ridx, tidx, hrows, rrows, trows, outv, sem):
        wid = lax.axis_index("s") * _NC + lax.axis_index("c")
        base = wid * _BPW

        pltpu.sync_copy(head_hbm.at[wid], hidx)
        pltpu.sync_copy(rel_hbm.at[wid], ridx)
        pltpu.sync_copy(tail_hbm.at[wid], tidx)

        copies = []
        for j in range(_NCH):
            dst = pl.ds(j * _CHK, _CHK)
            copies.append(pltpu.async_copy(ent_hbm.at[hidx.at[j]], hrows.at[dst], sem))
            copies.append(pltpu.async_copy(relw_hbm.at[ridx.at[j]], rrows.at[dst], sem))
            copies.append(pltpu.async_copy(ent_hbm.at[tidx.at[j]], trows.at[dst], sem))
        for c in copies:
            c.wait()

        row_iota = jnp.arange(16, dtype=jnp.int32)

        def body(g, carry):
            res = jnp.zeros((16,), jnp.float32)
            for rr in range(16):
                i = g * 16 + rr
                acc = jnp.zeros((16,), jnp.float32)
                for c in range(_EMBED_DIM // 16):
                    sl = pl.ds(c * 16, 16)
                    acc = acc + jnp.abs(hrows[i, sl] + rrows[i, sl] - trows[i, sl])
                res = jnp.where(row_iota == rr, jnp.sum(acc), res)
            outv[pl.ds(g * 16, 16)] = res
            return carry

        lax.fori_loop(0, _BPW // 16, body, 0)
        pltpu.sync_copy(outv, out_hbm.at[pl.ds(base, _BPW)])

    return trans_e


def kernel(head, relation, tail, entity_weight, relation_weight):
    fn = _build()
    h = head.reshape(_NW, _NCH, _CHK)
    r = relation.reshape(_NW, _NCH, _CHK)
    t = tail.reshape(_NW, _NCH, _CHK)
    return fn(h, r, t, entity_weight, relation_weight)
